# Initial kernel scaffold; baseline (speedup 1.0000x reference)
import jax
import jax.numpy as jnp
from jax.experimental import pallas as pl


def kernel(x, edge_index, W, b):
    raise NotImplementedError("write your pallas kernel here")


# speedup vs baseline: 13.4154x; 13.4154x over previous
import functools

import jax
import jax.numpy as jnp
from jax import lax
from jax.experimental import pallas as pl
from jax.experimental.pallas import tpu as pltpu
from jax.experimental.pallas import tpu_sc as plsc

N = 10000
D = 128
E = 320000

NC = 2
NS = 16
NW = NC * NS
CHUNK = 128
CPW = 80
GRP = 8
NGRP = CPW // GRP
E_PAD = NW * CPW * CHUNK
N_PAD = 10240
RPT = N_PAD // NS
DUMMY_ROW = N

ROW_BLK = 512


def _deg_body(rows_hbm, zeros_hbm, out_hbm, cnt, idx_v):
    c = lax.axis_index("c")
    s = lax.axis_index("s")
    wid = s * NC + c
    pltpu.sync_copy(zeros_hbm, cnt)
    pltpu.sync_copy(rows_hbm.at[wid], idx_v)
    ones = jnp.ones((16,), jnp.float32)

    def chunk(ci, carry):
        for j in range(CHUNK // 16):
            iv = idx_v[ci, pl.ds(j * 16, 16)]
            plsc.addupdate_scatter(cnt, [iv], ones)
        return carry

    lax.fori_loop(0, CPW, chunk, 0)
    pltpu.sync_copy(cnt, out_hbm.at[wid])


@jax.jit
def _deg_call(rows3, zeros1d):
    mesh = plsc.VectorSubcoreMesh(core_axis_name="c", subcore_axis_name="s")
    return pl.kernel(
        _deg_body,
        out_type=jax.ShapeDtypeStruct((NW, N_PAD), jnp.float32),
        mesh=mesh,
        scratch_types=[
            pltpu.VMEM((N_PAD,), jnp.float32),
            pltpu.VMEM((CPW, CHUNK), jnp.int32),
        ],
        compiler_params=pltpu.CompilerParams(needs_layout_passes=False),
    )(rows3, zeros1d)


def _msg_body(rows_hbm, cols_hbm, g_hbm, zeros_hbm, out_hbm,
              acc, ibuf_r, ibuf_c, msg0, msg1, sem0, sem1):
    c = lax.axis_index("c")
    s = lax.axis_index("s")
    wid = s * NC + c
    pltpu.sync_copy(zeros_hbm, acc.at[pl.ds(s * RPT, RPT)])
    plsc.subcore_barrier()

    def start(j, buf, sem):
        pltpu.async_copy(g_hbm.at[ibuf_c.at[j]], buf, sem)

    def wait(j, buf, sem):
        pltpu.make_async_copy(g_hbm.at[ibuf_c.at[j]], buf, sem).wait()

    def scatter(j, buf):
        pltpu.sync_copy(buf, acc.at[ibuf_r.at[j]], add=True)

    def group(g, carry):
        pltpu.sync_copy(rows_hbm.at[wid, pl.ds(g * GRP, GRP)], ibuf_r)
        pltpu.sync_copy(cols_hbm.at[wid, pl.ds(g * GRP, GRP)], ibuf_c)
        start(0, msg0, sem0)
        for j in range(0, GRP, 2):
            start(j + 1, msg1, sem1)
            wait(j, msg0, sem0)
            scatter(j, msg0)
            if j + 2 < GRP:
                start(j + 2, msg0, sem0)
            wait(j + 1, msg1, sem1)
            scatter(j + 1, msg1)
        return carry

    lax.fori_loop(0, NGRP, group, 0)
    plsc.subcore_barrier()
    pltpu.sync_copy(acc.at[pl.ds(s * RPT, RPT)],
                    out_hbm.at[c, pl.ds(s * RPT, RPT)])


@jax.jit
def _msg_call(rows3, cols3, g, zeros128):
    mesh = plsc.VectorSubcoreMesh(core_axis_name="c", subcore_axis_name="s")
    return pl.kernel(
        _msg_body,
        out_type=jax.ShapeDtypeStruct((NC, N_PAD, D), jnp.float32),
        mesh=mesh,
        scratch_types=[
            pltpu.VMEM_SHARED((N_PAD, D), jnp.float32),
            pltpu.VMEM((GRP, CHUNK), jnp.int32),
            pltpu.VMEM((GRP, CHUNK), jnp.int32),
            pltpu.VMEM((CHUNK, D), jnp.float32),
            pltpu.VMEM((CHUNK, D), jnp.float32),
            pltpu.SemaphoreType.DMA,
            pltpu.SemaphoreType.DMA,
        ],
    )(rows3, cols3, g, zeros128)


def _g_body(x_ref, w_ref, b_ref, degp_ref, g_ref):
    h = lax.dot_general(x_ref[...], w_ref[...], (((1,), (1,)), ((), ())),
                        preferred_element_type=jnp.float32)
    h = h + b_ref[...]
    deg = jnp.sum(degp_ref[...], axis=0) + 1.0
    dis = lax.rsqrt(deg)
    g_ref[...] = h * dis[:, None]


@jax.jit
def _g_call(x_pad, W, b2, degp):
    grid = N_PAD // ROW_BLK
    return pl.pallas_call(
        _g_body,
        grid=(grid,),
        in_specs=[
            pl.BlockSpec((ROW_BLK, D), lambda i: (i, 0)),
            pl.BlockSpec((D, D), lambda i: (0, 0)),
            pl.BlockSpec((1, D), lambda i: (0, 0)),
            pl.BlockSpec((NW, ROW_BLK), lambda i: (0, i)),
        ],
        out_specs=pl.BlockSpec((ROW_BLK, D), lambda i: (i, 0)),
        out_shape=jax.ShapeDtypeStruct((N_PAD, D), jnp.float32),
    )(x_pad, W, b2, degp)


def _out_body(p_ref, g_ref, degp_ref, o_ref):
    deg = jnp.sum(degp_ref[...], axis=0) + 1.0
    dis = lax.rsqrt(deg)
    o_ref[...] = (p_ref[0] + p_ref[1] + g_ref[...]) * dis[:, None]


@jax.jit
def _out_call(p, g, degp):
    grid = N_PAD // ROW_BLK
    return pl.pallas_call(
        _out_body,
        grid=(grid,),
        in_specs=[
            pl.BlockSpec((NC, ROW_BLK, D), lambda i: (0, i, 0)),
            pl.BlockSpec((ROW_BLK, D), lambda i: (i, 0)),
            pl.BlockSpec((NW, ROW_BLK), lambda i: (0, i)),
        ],
        out_specs=pl.BlockSpec((ROW_BLK, D), lambda i: (i, 0)),
        out_shape=jax.ShapeDtypeStruct((N_PAD, D), jnp.float32),
    )(p, g, degp)


def kernel(x, edge_index, W, b):
    row = edge_index[0].astype(jnp.int32)
    col = edge_index[1].astype(jnp.int32)
    pad = E_PAD - E
    rows3 = jnp.concatenate(
        [row, jnp.full((pad,), DUMMY_ROW, jnp.int32)]).reshape(NW, CPW, CHUNK)
    cols3 = jnp.concatenate(
        [col, jnp.zeros((pad,), jnp.int32)]).reshape(NW, CPW, CHUNK)
    zeros1d = jnp.zeros((N_PAD,), jnp.float32)
    zeros128 = jnp.zeros((RPT, D), jnp.float32)
    x_pad = jnp.pad(x, ((0, N_PAD - N), (0, 0)))

    degp = _deg_call(rows3, zeros1d)
    g = _g_call(x_pad, W, b.reshape(1, D), degp)
    p = _msg_call(rows3, cols3, g, zeros128)
    return _out_call(p, g, degp)[:N]


# speedup vs baseline: 37.0998x; 2.7655x over previous
import functools

import jax
import jax.numpy as jnp
from jax import lax
from jax.experimental import pallas as pl
from jax.experimental.pallas import tpu as pltpu
from jax.experimental.pallas import tpu_sc as plsc

N = 10000
D = 128
E = 320000

NC = 2
NS = 16
NW = NC * NS
CHUNK = 128
CPW = 80
GRP = 8
NGRP = CPW // GRP
E_PAD = NW * CPW * CHUNK
N_PAD = 10240
RPT = N_PAD // NS
DUMMY_ROW = N

ROW_BLK = 512


def _deg_body(rows_hbm, zeros_hbm, out_hbm, cnt, idx_v):
    c = lax.axis_index("c")
    s = lax.axis_index("s")
    wid = s * NC + c
    pltpu.sync_copy(zeros_hbm, cnt)
    pltpu.sync_copy(rows_hbm.at[wid], idx_v)
    ones = jnp.ones((16,), jnp.float32)

    def chunk(ci, carry):
        for j in range(CHUNK // 16):
            iv = idx_v[ci, pl.ds(j * 16, 16)]
            plsc.addupdate_scatter(cnt, [iv], ones)
        return carry

    lax.fori_loop(0, CPW, chunk, 0)
    pltpu.sync_copy(cnt, out_hbm.at[wid])


@jax.jit
def _deg_call(rows3, zeros1d):
    mesh = plsc.VectorSubcoreMesh(core_axis_name="c", subcore_axis_name="s")
    return pl.kernel(
        _deg_body,
        out_type=jax.ShapeDtypeStruct((NW, N_PAD), jnp.float32),
        mesh=mesh,
        scratch_types=[
            pltpu.VMEM((N_PAD,), jnp.float32),
            pltpu.VMEM((CPW, CHUNK), jnp.int32),
        ],
        compiler_params=pltpu.CompilerParams(needs_layout_passes=False),
    )(rows3, zeros1d)


def _msg_body(rows_hbm, cols_hbm, g_hbm, zeros_hbm, out_hbm,
              acc, ibuf_r, ibuf_c, msg0, msg1, sem0, sem1):
    c = lax.axis_index("c")
    s = lax.axis_index("s")
    wid = s * NC + c
    pltpu.sync_copy(zeros_hbm, acc.at[pl.ds(s * RPT, RPT)])
    plsc.subcore_barrier()

    def start(j, buf, sem):
        pltpu.async_copy(g_hbm.at[ibuf_c.at[j]], buf, sem)

    def wait(j, buf, sem):
        pltpu.make_async_copy(g_hbm.at[ibuf_c.at[j]], buf, sem).wait()

    def scatter(j, buf):
        pltpu.sync_copy(buf, acc.at[ibuf_r.at[j]], add=True)

    def group(g, carry):
        pltpu.sync_copy(rows_hbm.at[wid, pl.ds(g * GRP, GRP)], ibuf_r)
        pltpu.sync_copy(cols_hbm.at[wid, pl.ds(g * GRP, GRP)], ibuf_c)
        start(0, msg0, sem0)
        for j in range(0, GRP, 2):
            start(j + 1, msg1, sem1)
            wait(j, msg0, sem0)
            scatter(j, msg0)
            if j + 2 < GRP:
                start(j + 2, msg0, sem0)
            wait(j + 1, msg1, sem1)
            scatter(j + 1, msg1)
        return carry

    lax.fori_loop(0, NGRP, group, 0)
    plsc.subcore_barrier()
    pltpu.sync_copy(acc.at[pl.ds(s * RPT, RPT)],
                    out_hbm.at[c, pl.ds(s * RPT, RPT)])


@jax.jit
def _msg_call(rows3, cols3, g, zeros128):
    mesh = plsc.VectorSubcoreMesh(core_axis_name="c", subcore_axis_name="s")
    return pl.kernel(
        _msg_body,
        out_type=jax.ShapeDtypeStruct((NC, N_PAD, D), jnp.float32),
        mesh=mesh,
        scratch_types=[
            pltpu.VMEM_SHARED((N_PAD, D), jnp.float32),
            pltpu.VMEM((GRP, CHUNK), jnp.int32),
            pltpu.VMEM((GRP, CHUNK), jnp.int32),
            pltpu.VMEM((CHUNK, D), jnp.float32),
            pltpu.VMEM((CHUNK, D), jnp.float32),
            pltpu.SemaphoreType.DMA,
            pltpu.SemaphoreType.DMA,
        ],
    )(rows3, cols3, g, zeros128)


def _g_body(x_ref, w_ref, b_ref, degp_ref, g_ref):
    h = lax.dot_general(x_ref[...], w_ref[...], (((1,), (1,)), ((), ())),
                        preferred_element_type=jnp.float32)
    h = h + b_ref[...]
    deg = jnp.sum(degp_ref[...], axis=0) + 1.0
    dis = lax.rsqrt(deg)
    g_ref[...] = h * dis[:, None]


@jax.jit
def _g_call(x_pad, W, b2, degp):
    grid = N_PAD // ROW_BLK
    return pl.pallas_call(
        _g_body,
        grid=(grid,),
        in_specs=[
            pl.BlockSpec((ROW_BLK, D), lambda i: (i, 0)),
            pl.BlockSpec((D, D), lambda i: (0, 0)),
            pl.BlockSpec((1, D), lambda i: (0, 0)),
            pl.BlockSpec((NW, ROW_BLK), lambda i: (0, i)),
        ],
        out_specs=pl.BlockSpec((ROW_BLK, D), lambda i: (i, 0)),
        out_shape=jax.ShapeDtypeStruct((N_PAD, D), jnp.float32),
    )(x_pad, W, b2, degp)


def _out_body(p_ref, g_ref, degp_ref, o_ref):
    deg = jnp.sum(degp_ref[...], axis=0) + 1.0
    dis = lax.rsqrt(deg)
    o_ref[...] = (p_ref[0] + p_ref[1] + g_ref[...]) * dis[:, None]


@jax.jit
def _out_call(p, g, degp):
    grid = N_PAD // ROW_BLK
    return pl.pallas_call(
        _out_body,
        grid=(grid,),
        in_specs=[
            pl.BlockSpec((NC, ROW_BLK, D), lambda i: (0, i, 0)),
            pl.BlockSpec((ROW_BLK, D), lambda i: (i, 0)),
            pl.BlockSpec((NW, ROW_BLK), lambda i: (0, i)),
        ],
        out_specs=pl.BlockSpec((ROW_BLK, D), lambda i: (i, 0)),
        out_shape=jax.ShapeDtypeStruct((N_PAD, D), jnp.float32),
    )(p, g, degp)


def kernel(x, edge_index, W, b):
    row = edge_index[0].astype(jnp.int32)
    col = edge_index[1].astype(jnp.int32)
    pad = E_PAD - E
    spread = jnp.arange(pad, dtype=jnp.int32) % 128
    rows3 = jnp.concatenate([row, DUMMY_ROW + spread]).reshape(NW, CPW, CHUNK)
    cols3 = jnp.concatenate([col, spread]).reshape(NW, CPW, CHUNK)
    zeros1d = jnp.zeros((N_PAD,), jnp.float32)
    zeros128 = jnp.zeros((RPT, D), jnp.float32)
    x_pad = jnp.pad(x, ((0, N_PAD - N), (0, 0)))

    degp = _deg_call(rows3, zeros1d)
    g = _g_call(x_pad, W, b.reshape(1, D), degp)
    p = _msg_call(rows3, cols3, g, zeros128)
    return _out_call(p, g, degp)[:N]


# speedup vs baseline: 39.7466x; 1.0713x over previous
import functools

import jax
import jax.numpy as jnp
from jax import lax
from jax.experimental import pallas as pl
from jax.experimental.pallas import tpu as pltpu
from jax.experimental.pallas import tpu_sc as plsc

N = 10000
D = 128
E = 320000

NC = 2
NS = 16
NW = NC * NS
CHUNK = 128
CPW = 80
GRP = 16
NGRP = CPW // GRP
E_PAD = NW * CPW * CHUNK
N_PAD = 10240
RPT = N_PAD // NS
DUMMY_ROW = N

ROW_BLK = 512


def _deg_body(rows_hbm, zeros_hbm, out_hbm, cnt, idx_v):
    c = lax.axis_index("c")
    s = lax.axis_index("s")
    wid = s * NC + c
    pltpu.sync_copy(zeros_hbm, cnt)
    pltpu.sync_copy(rows_hbm.at[wid], idx_v)
    ones = jnp.ones((16,), jnp.float32)

    def chunk(ci, carry):
        for j in range(CHUNK // 16):
            iv = idx_v[ci, pl.ds(j * 16, 16)]
            plsc.addupdate_scatter(cnt, [iv], ones)
        return carry

    lax.fori_loop(0, CPW, chunk, 0)
    pltpu.sync_copy(cnt, out_hbm.at[wid])


@jax.jit
def _deg_call(rows3, zeros1d):
    mesh = plsc.VectorSubcoreMesh(core_axis_name="c", subcore_axis_name="s")
    return pl.kernel(
        _deg_body,
        out_type=jax.ShapeDtypeStruct((NW, N_PAD), jnp.float32),
        mesh=mesh,
        scratch_types=[
            pltpu.VMEM((N_PAD,), jnp.float32),
            pltpu.VMEM((CPW, CHUNK), jnp.int32),
        ],
        compiler_params=pltpu.CompilerParams(needs_layout_passes=False),
    )(rows3, zeros1d)


def _msg_body(rows_hbm, cols_hbm, g_hbm, zeros_hbm, out_hbm,
              acc, ibuf_r, ibuf_c, msg0, msg1, sem0, sem1):
    c = lax.axis_index("c")
    s = lax.axis_index("s")
    wid = s * NC + c
    pltpu.sync_copy(zeros_hbm, acc.at[pl.ds(s * RPT, RPT)])
    plsc.subcore_barrier()

    def start(j, buf, sem):
        pltpu.async_copy(g_hbm.at[ibuf_c.at[j]], buf, sem)

    def wait(j, buf, sem):
        pltpu.make_async_copy(g_hbm.at[ibuf_c.at[j]], buf, sem).wait()

    def scatter(j, buf):
        pltpu.sync_copy(buf, acc.at[ibuf_r.at[j]], add=True)

    def group(g, carry):
        pltpu.sync_copy(rows_hbm.at[wid, pl.ds(g * GRP, GRP)], ibuf_r)
        pltpu.sync_copy(cols_hbm.at[wid, pl.ds(g * GRP, GRP)], ibuf_c)
        start(0, msg0, sem0)
        for j in range(0, GRP, 2):
            start(j + 1, msg1, sem1)
            wait(j, msg0, sem0)
            scatter(j, msg0)
            if j + 2 < GRP:
                start(j + 2, msg0, sem0)
            wait(j + 1, msg1, sem1)
            scatter(j + 1, msg1)
        return carry

    lax.fori_loop(0, NGRP, group, 0)
    plsc.subcore_barrier()
    pltpu.sync_copy(acc.at[pl.ds(s * RPT, RPT)],
                    out_hbm.at[c, pl.ds(s * RPT, RPT)])


@jax.jit
def _msg_call(rows3, cols3, g, zeros128):
    mesh = plsc.VectorSubcoreMesh(core_axis_name="c", subcore_axis_name="s")
    return pl.kernel(
        _msg_body,
        out_type=jax.ShapeDtypeStruct((NC, N_PAD, D), jnp.float32),
        mesh=mesh,
        scratch_types=[
            pltpu.VMEM_SHARED((N_PAD, D), jnp.float32),
            pltpu.VMEM((GRP, CHUNK), jnp.int32),
            pltpu.VMEM((GRP, CHUNK), jnp.int32),
            pltpu.VMEM((CHUNK, D), jnp.float32),
            pltpu.VMEM((CHUNK, D), jnp.float32),
            pltpu.SemaphoreType.DMA,
            pltpu.SemaphoreType.DMA,
        ],
    )(rows3, cols3, g, zeros128)


def _g_body(x_ref, w_ref, b_ref, degp_ref, g_ref):
    h = lax.dot_general(x_ref[...], w_ref[...], (((1,), (1,)), ((), ())),
                        preferred_element_type=jnp.float32)
    h = h + b_ref[...]
    deg = jnp.sum(degp_ref[...], axis=0) + 1.0
    dis = lax.rsqrt(deg)
    g_ref[...] = h * dis[:, None]


@jax.jit
def _g_call(x, W, b2, degp):
    grid = N_PAD // ROW_BLK
    return pl.pallas_call(
        _g_body,
        grid=(grid,),
        in_specs=[
            pl.BlockSpec((ROW_BLK, D), lambda i: (i, 0)),
            pl.BlockSpec((D, D), lambda i: (0, 0)),
            pl.BlockSpec((1, D), lambda i: (0, 0)),
            pl.BlockSpec((NW, ROW_BLK), lambda i: (0, i)),
        ],
        out_specs=pl.BlockSpec((ROW_BLK, D), lambda i: (i, 0)),
        out_shape=jax.ShapeDtypeStruct((N_PAD, D), jnp.float32),
    )(x, W, b2, degp)


def _out_body(p_ref, g_ref, degp_ref, o_ref):
    deg = jnp.sum(degp_ref[...], axis=0) + 1.0
    dis = lax.rsqrt(deg)
    o_ref[...] = (p_ref[0] + p_ref[1] + g_ref[...]) * dis[:, None]


@jax.jit
def _out_call(p, g, degp):
    grid = N_PAD // ROW_BLK
    return pl.pallas_call(
        _out_body,
        grid=(grid,),
        in_specs=[
            pl.BlockSpec((NC, ROW_BLK, D), lambda i: (0, i, 0)),
            pl.BlockSpec((ROW_BLK, D), lambda i: (i, 0)),
            pl.BlockSpec((NW, ROW_BLK), lambda i: (0, i)),
        ],
        out_specs=pl.BlockSpec((ROW_BLK, D), lambda i: (i, 0)),
        out_shape=jax.ShapeDtypeStruct((N_PAD, D), jnp.float32),
    )(p, g, degp)


def kernel(x, edge_index, W, b):
    row = edge_index[0].astype(jnp.int32)
    col = edge_index[1].astype(jnp.int32)
    pad = E_PAD - E
    spread = jnp.arange(pad, dtype=jnp.int32) % 128
    rows3 = jnp.concatenate([row, DUMMY_ROW + spread]).reshape(NW, CPW, CHUNK)
    cols3 = jnp.concatenate([col, spread]).reshape(NW, CPW, CHUNK)
    zeros1d = jnp.zeros((N_PAD,), jnp.float32)
    zeros128 = jnp.zeros((RPT, D), jnp.float32)

    degp = _deg_call(rows3, zeros1d)
    g = _g_call(x, W, b.reshape(1, D), degp)
    p = _msg_call(rows3, cols3, g, zeros128)
    return _out_call(p, g, degp)[:N]


# speedup vs baseline: 42.1121x; 1.0595x over previous
import functools

import jax
import jax.numpy as jnp
from jax import lax
from jax.experimental import pallas as pl
from jax.experimental.pallas import tpu as pltpu
from jax.experimental.pallas import tpu_sc as plsc

N = 10000
D = 128
E = 320000

NC = 2
NS = 16
NW = NC * NS
CHUNK = 128
CPW = 80
GRP = 4
NGRP = CPW // GRP
NB = NGRP // 2
E_PAD = NW * CPW * CHUNK
N_PAD = 10240
RPT = N_PAD // NS
DUMMY_ROW = N

ROW_BLK = 512


def _deg_body(rows_hbm, zeros_hbm, out_hbm, cnt, idx_v):
    c = lax.axis_index("c")
    s = lax.axis_index("s")
    wid = s * NC + c
    pltpu.sync_copy(zeros_hbm, cnt)
    pltpu.sync_copy(rows_hbm.at[wid], idx_v)
    ones = jnp.ones((16,), jnp.float32)

    def chunk(ci, carry):
        for j in range(CHUNK // 16):
            iv = idx_v[ci, pl.ds(j * 16, 16)]
            plsc.addupdate_scatter(cnt, [iv], ones)
        return carry

    lax.fori_loop(0, CPW, chunk, 0)
    pltpu.sync_copy(cnt, out_hbm.at[wid])


@jax.jit
def _deg_call(rows3, zeros1d):
    mesh = plsc.VectorSubcoreMesh(core_axis_name="c", subcore_axis_name="s")
    return pl.kernel(
        _deg_body,
        out_type=jax.ShapeDtypeStruct((NW, N_PAD), jnp.float32),
        mesh=mesh,
        scratch_types=[
            pltpu.VMEM((N_PAD,), jnp.float32),
            pltpu.VMEM((CPW, CHUNK), jnp.int32),
        ],
        compiler_params=pltpu.CompilerParams(needs_layout_passes=False),
    )(rows3, zeros1d)


def _msg_body(rows_hbm, cols_hbm, g_hbm, zeros_hbm, out_hbm,
              acc, ibA_r, ibA_c, ibB_r, ibB_c, msg0, msg1,
              sem0, sem1, semA, semB):
    c = lax.axis_index("c")
    s = lax.axis_index("s")
    wid = s * NC + c
    pltpu.sync_copy(zeros_hbm, acc.at[pl.ds(s * RPT, RPT)])
    plsc.subcore_barrier()

    def start(cref, buf, sem):
        pltpu.async_copy(g_hbm.at[cref], buf, sem)

    def wait(buf, sem):
        pltpu.make_async_copy(g_hbm.at[ibA_c.at[0]], buf, sem).wait()

    def scatter(rref, buf):
        pltpu.sync_copy(buf, acc.at[rref], add=True)

    def fetch_idx(g, rbuf, cbuf, sem):
        pltpu.async_copy(rows_hbm.at[wid, pl.ds(g * GRP, GRP)], rbuf, sem)
        pltpu.async_copy(cols_hbm.at[wid, pl.ds(g * GRP, GRP)], cbuf, sem)

    def wait_idx(g, rbuf, cbuf, sem):
        pltpu.make_async_copy(rows_hbm.at[wid, pl.ds(g * GRP, GRP)], rbuf,
                              sem).wait()
        pltpu.make_async_copy(cols_hbm.at[wid, pl.ds(g * GRP, GRP)], cbuf,
                              sem).wait()

    fetch_idx(0, ibA_r, ibA_c, semA)
    wait_idx(0, ibA_r, ibA_c, semA)
    fetch_idx(1, ibB_r, ibB_c, semB)
    start(ibA_c.at[0], msg0, sem0)

    def body(i, carry):
        ga = lax.rem(2 * i + 2, NGRP)
        gb = lax.rem(2 * i + 3, NGRP)
        start(ibA_c.at[1], msg1, sem1)
        wait(msg0, sem0)
        scatter(ibA_r.at[0], msg0)
        start(ibA_c.at[2], msg0, sem0)
        wait(msg1, sem1)
        scatter(ibA_r.at[1], msg1)
        start(ibA_c.at[3], msg1, sem1)
        wait(msg0, sem0)
        scatter(ibA_r.at[2], msg0)
        wait_idx(gb, ibB_r, ibB_c, semB)
        start(ibB_c.at[0], msg0, sem0)
        wait(msg1, sem1)
        scatter(ibA_r.at[3], msg1)
        fetch_idx(ga, ibA_r, ibA_c, semA)
        start(ibB_c.at[1], msg1, sem1)
        wait(msg0, sem0)
        scatter(ibB_r.at[0], msg0)
        start(ibB_c.at[2], msg0, sem0)
        wait(msg1, sem1)
        scatter(ibB_r.at[1], msg1)
        start(ibB_c.at[3], msg1, sem1)
        wait(msg0, sem0)
        scatter(ibB_r.at[2], msg0)
        wait_idx(ga, ibA_r, ibA_c, semA)
        start(ibA_c.at[0], msg0, sem0)
        wait(msg1, sem1)
        scatter(ibB_r.at[3], msg1)
        fetch_idx(gb, ibB_r, ibB_c, semB)
        return carry

    lax.fori_loop(0, NB, body, 0)
    wait(msg0, sem0)
    wait_idx(1, ibB_r, ibB_c, semB)
    plsc.subcore_barrier()
    pltpu.sync_copy(acc.at[pl.ds(s * RPT, RPT)],
                    out_hbm.at[c, pl.ds(s * RPT, RPT)])


@jax.jit
def _msg_call(rows3, cols3, g, zeros128):
    mesh = plsc.VectorSubcoreMesh(core_axis_name="c", subcore_axis_name="s")
    return pl.kernel(
        _msg_body,
        out_type=jax.ShapeDtypeStruct((NC, N_PAD, D), jnp.float32),
        mesh=mesh,
        scratch_types=[
            pltpu.VMEM_SHARED((N_PAD, D), jnp.float32),
            pltpu.VMEM((GRP, CHUNK), jnp.int32),
            pltpu.VMEM((GRP, CHUNK), jnp.int32),
            pltpu.VMEM((GRP, CHUNK), jnp.int32),
            pltpu.VMEM((GRP, CHUNK), jnp.int32),
            pltpu.VMEM((CHUNK, D), jnp.float32),
            pltpu.VMEM((CHUNK, D), jnp.float32),
            pltpu.SemaphoreType.DMA,
            pltpu.SemaphoreType.DMA,
            pltpu.SemaphoreType.DMA,
            pltpu.SemaphoreType.DMA,
        ],
    )(rows3, cols3, g, zeros128)


def _g_body(x_ref, w_ref, b_ref, degp_ref, g_ref):
    h = lax.dot_general(x_ref[...], w_ref[...], (((1,), (1,)), ((), ())),
                        preferred_element_type=jnp.float32)
    h = h + b_ref[...]
    deg = jnp.sum(degp_ref[...], axis=0) + 1.0
    dis = lax.rsqrt(deg)
    g_ref[...] = h * dis[:, None]


@jax.jit
def _g_call(x, W, b2, degp):
    grid = N_PAD // ROW_BLK
    return pl.pallas_call(
        _g_body,
        grid=(grid,),
        in_specs=[
            pl.BlockSpec((ROW_BLK, D), lambda i: (i, 0)),
            pl.BlockSpec((D, D), lambda i: (0, 0)),
            pl.BlockSpec((1, D), lambda i: (0, 0)),
            pl.BlockSpec((NW, ROW_BLK), lambda i: (0, i)),
        ],
        out_specs=pl.BlockSpec((ROW_BLK, D), lambda i: (i, 0)),
        out_shape=jax.ShapeDtypeStruct((N_PAD, D), jnp.float32),
    )(x, W, b2, degp)


def _out_body(p_ref, g_ref, degp_ref, o_ref):
    deg = jnp.sum(degp_ref[...], axis=0) + 1.0
    dis = lax.rsqrt(deg)
    o_ref[...] = (p_ref[0] + p_ref[1] + g_ref[...]) * dis[:, None]


@jax.jit
def _out_call(p, g, degp):
    grid = N_PAD // ROW_BLK
    return pl.pallas_call(
        _out_body,
        grid=(grid,),
        in_specs=[
            pl.BlockSpec((NC, ROW_BLK, D), lambda i: (0, i, 0)),
            pl.BlockSpec((ROW_BLK, D), lambda i: (i, 0)),
            pl.BlockSpec((NW, ROW_BLK), lambda i: (0, i)),
        ],
        out_specs=pl.BlockSpec((ROW_BLK, D), lambda i: (i, 0)),
        out_shape=jax.ShapeDtypeStruct((N_PAD, D), jnp.float32),
    )(p, g, degp)


def kernel(x, edge_index, W, b):
    row = edge_index[0].astype(jnp.int32)
    col = edge_index[1].astype(jnp.int32)
    pad = E_PAD - E
    spread = jnp.arange(pad, dtype=jnp.int32) % 128
    rows3 = jnp.concatenate([row, DUMMY_ROW + spread]).reshape(NW, CPW, CHUNK)
    cols3 = jnp.concatenate([col, spread]).reshape(NW, CPW, CHUNK)
    zeros1d = jnp.zeros((N_PAD,), jnp.float32)
    zeros128 = jnp.zeros((RPT, D), jnp.float32)

    degp = _deg_call(rows3, zeros1d)
    g = _g_call(x, W, b.reshape(1, D), degp)
    p = _msg_call(rows3, cols3, g, zeros128)
    return _out_call(p, g, degp)[:N]
